# baseline (speedup 1.0000x reference)
import jax
import jax.numpy as jnp
import numpy as np
from jax.experimental import pallas as pl
from jax.experimental.pallas import tpu as pltpu

K_TOP_N = 16000
HI = jax.lax.Precision.HIGHEST

H, Wd, Z, CH = 50, 50, 8, 256
H2, W2, Z2 = 2 * H, 2 * Wd, 2 * Z
NVOX = H2 * W2 * Z2
ABLK = 4
GRID1 = H2 // ABLK
ROWS_PER_BLK = ABLK * W2 * Z2


def _upmat(n: int) -> np.ndarray:
    A = np.zeros((2 * n, n), dtype=np.float32)
    for i in range(2 * n):
        c = i / 2 - 0.25
        jlo = int(np.floor(c))
        frac = np.float32(c - jlo)
        if jlo < 0:
            A[i, 0] = 1.0
        elif jlo + 1 >= n:
            A[i, n - 1] = 1.0
        else:
            A[i, jlo] = np.float32(1.0) - frac
            A[i, jlo + 1] = frac
    return A

_A_H = _upmat(H)
_A_W = _upmat(Wd)
_A_Z = _upmat(Z)


CC = 32
NCHUNK = CH // CC


def _occ_body(ah_ref, vt_ref, aw_ref, az_ref, w_ref, b_ref, o_ref, f_scr):
    ci = pl.program_id(1)
    ah = ah_ref[...].reshape(ABLK, H)
    e1 = jax.lax.dot_general(ah, vt_ref[...], (((1,), (0,)), ((), ())),
                             precision=HI, preferred_element_type=jnp.float32)
    e1t = jnp.swapaxes(e1.reshape(ABLK, CC, Wd, Z), -1, -2)
    e2 = jax.lax.dot_general(e1t.reshape(ABLK * CC * Z, Wd), aw_ref[...],
                             (((1,), (1,)), ((), ())),
                             precision=HI, preferred_element_type=jnp.float32)
    e2t = jnp.swapaxes(e2.reshape(ABLK * CC, Z, W2), -1, -2)
    e3 = jax.lax.dot_general(e2t.reshape(ABLK * CC * W2, Z), az_ref[...],
                             (((1,), (1,)), ((), ())),
                             precision=HI, preferred_element_type=jnp.float32)
    fb = e3.astype(jnp.bfloat16).reshape(ABLK, CC, W2, Z2)
    ft = jnp.transpose(fb, (0, 2, 3, 1)).reshape(ABLK * W2 * Z2, CC)
    f_scr[pl.ds(ci * CC, CC), :] = ft.T

    @pl.when(ci == NCHUNK - 1)
    def _head():
        feats = f_scr[...].T
        y = jax.lax.dot_general(feats, w_ref[...], (((1,), (0,)), ((), ())),
                                preferred_element_type=jnp.float32)
        o_ref[...] = 1.0 / (1.0 + jnp.exp(-(y[:, 0:1] + b_ref[0, 0])))


def _select_body(occ_ref, mask_ref, valid_ref, occo_ref):
    x = occ_ref[...]
    occo_ref[...] = x
    keys = jax.lax.bitcast_convert_type(x, jnp.int32)
    idx = (jax.lax.broadcasted_iota(jnp.int32, x.shape, 0) * x.shape[1]
           + jax.lax.broadcasted_iota(jnp.int32, x.shape, 1))

    def val_step(_, lohi):
        lo, hi = lohi
        mid = lo + (hi - lo + 1) // 2
        ge = jnp.sum((keys >= mid).astype(jnp.int32)) >= K_TOP_N
        return (jnp.where(ge, mid, lo), jnp.where(ge, hi, mid - 1))

    lo0 = jnp.int32(0)
    hi0 = jnp.int32(0x3F800000)
    t, _ = jax.lax.fori_loop(0, 31, val_step, (lo0, hi0))

    n_gt = jnp.sum((keys > t).astype(jnp.int32))
    need = K_TOP_N - n_gt

    eq = keys == t

    def idx_step(_, lohi):
        lo, hi = lohi
        mid = lo + (hi - lo) // 2
        cnt = jnp.sum((eq & (idx <= mid)).astype(jnp.int32))
        ok = cnt >= need
        return (jnp.where(ok, lo, mid + 1), jnp.where(ok, mid, hi))

    jstar, _ = jax.lax.fori_loop(0, 18, idx_step,
                                 (jnp.int32(0), jnp.int32(NVOX - 1)))

    m = (keys > t) | (eq & (idx <= jstar))
    mask_ref[...] = m.astype(jnp.float32)
    valid_ref[...] = m.astype(jnp.int32)


@jax.jit
def kernel(volume, W, b):
    vt = jnp.transpose(volume[0], (1, 0, 2, 3)).reshape(H, CH * Wd * Z)
    wpad = jnp.zeros((CH, 128), jnp.bfloat16).at[:, 0].set(W[:, 0].astype(jnp.bfloat16))
    b2 = b.reshape(1, 1)

    occ_col = pl.pallas_call(
        _occ_body,
        grid=(GRID1, NCHUNK),
        in_specs=[
            pl.BlockSpec((1, ABLK, H), lambda i, c: (i, 0, 0)),
            pl.BlockSpec((H, CC * Wd * Z), lambda i, c: (0, c)),
            pl.BlockSpec((W2, Wd), lambda i, c: (0, 0)),
            pl.BlockSpec((Z2, Z), lambda i, c: (0, 0)),
            pl.BlockSpec((CH, 128), lambda i, c: (0, 0)),
            pl.BlockSpec(memory_space=pltpu.SMEM),
        ],
        out_specs=pl.BlockSpec((ROWS_PER_BLK, 1), lambda i, c: (i, 0)),
        out_shape=jax.ShapeDtypeStruct((NVOX, 1), jnp.float32),
        scratch_shapes=[pltpu.VMEM((CH, ROWS_PER_BLK), jnp.bfloat16)],
        compiler_params=pltpu.CompilerParams(
            dimension_semantics=("arbitrary", "arbitrary"),
        ),
    )(jnp.asarray(_A_H).reshape(GRID1, ABLK, H), vt,
      jnp.asarray(_A_W), jnp.asarray(_A_Z), wpad, b2)

    occv = occ_col.reshape(NVOX // 128, 128)
    mask2, valid2, occ2 = pl.pallas_call(
        _select_body,
        grid=(1,),
        in_specs=[
            pl.BlockSpec((NVOX // 128, 128), lambda i: (0, 0)),
        ],
        out_specs=[
            pl.BlockSpec((NVOX // 128, 128), lambda i: (0, 0)),
            pl.BlockSpec((NVOX // 128, 128), lambda i: (0, 0)),
            pl.BlockSpec((NVOX // 128, 128), lambda i: (0, 0)),
        ],
        out_shape=[
            jax.ShapeDtypeStruct((NVOX // 128, 128), jnp.float32),
            jax.ShapeDtypeStruct((NVOX // 128, 128), jnp.int32),
            jax.ShapeDtypeStruct((NVOX // 128, 128), jnp.float32),
        ],
    )(occv)

    occ_preds = occ2.reshape(1, NVOX)
    mask_hard = mask2.reshape(1, NVOX)
    valid = valid2.reshape(1, 1, H2, W2, Z2)
    return (occ_preds, mask_hard, valid)
